# baseline (speedup 1.0000x reference)
import jax
import jax.numpy as jnp
from jax.experimental import pallas as pl
from jax.experimental.pallas import tpu as pltpu

_N_CHUNKS = 32


def _staged_copy(table_hbm, out_hbm, buf, sem_in, sem_out):
    rows = table_hbm.shape[0]
    chunk = rows // _N_CHUNKS

    def cin(i):
        return pltpu.make_async_copy(
            table_hbm.at[pl.ds(i * chunk, chunk)],
            buf.at[pl.ds(i * chunk, chunk)],
            sem_in.at[i],
        )

    def cout(i):
        return pltpu.make_async_copy(
            buf.at[pl.ds(i * chunk, chunk)],
            out_hbm.at[pl.ds(i * chunk, chunk)],
            sem_out.at[i],
        )

    for i in range(_N_CHUNKS):
        cin(i).start()
    for i in range(_N_CHUNKS):
        cin(i).wait()
        cout(i).start()
    for i in range(_N_CHUNKS):
        cout(i).wait()


def kernel(input_indices, position_embedding_table):
    seq_len = input_indices.shape[1]
    n_rows, dim = position_embedding_table.shape
    return pl.pallas_call(
        _staged_copy,
        in_specs=[pl.BlockSpec(memory_space=pltpu.HBM)],
        out_specs=pl.BlockSpec(memory_space=pltpu.HBM),
        out_shape=jax.ShapeDtypeStruct((seq_len, dim), position_embedding_table.dtype),
        scratch_shapes=[
            pltpu.VMEM((seq_len, dim), position_embedding_table.dtype),
            pltpu.SemaphoreType.DMA((_N_CHUNKS,)),
            pltpu.SemaphoreType.DMA((_N_CHUNKS,)),
        ],
    )(position_embedding_table)


# speedup vs baseline: 1.0433x; 1.0433x over previous
import jax
import jax.numpy as jnp
from jax.experimental import pallas as pl
from jax.experimental.pallas import tpu as pltpu

_N_CHUNKS = 8


def _staged_copy(table_hbm, out_hbm, buf, sem_in, sem_out):
    rows = table_hbm.shape[0]
    chunk = rows // _N_CHUNKS

    def cin(i):
        return pltpu.make_async_copy(
            table_hbm.at[pl.ds(i * chunk, chunk)],
            buf.at[pl.ds(i * chunk, chunk)],
            sem_in.at[i],
        )

    def cout(i):
        return pltpu.make_async_copy(
            buf.at[pl.ds(i * chunk, chunk)],
            out_hbm.at[pl.ds(i * chunk, chunk)],
            sem_out.at[i],
        )

    for i in range(_N_CHUNKS):
        cin(i).start()
    for i in range(_N_CHUNKS):
        cin(i).wait()
        cout(i).start()
    for i in range(_N_CHUNKS):
        cout(i).wait()


def kernel(input_indices, position_embedding_table):
    seq_len = input_indices.shape[1]
    n_rows, dim = position_embedding_table.shape
    return pl.pallas_call(
        _staged_copy,
        in_specs=[pl.BlockSpec(memory_space=pltpu.HBM)],
        out_specs=pl.BlockSpec(memory_space=pltpu.HBM),
        out_shape=jax.ShapeDtypeStruct((seq_len, dim), position_embedding_table.dtype),
        scratch_shapes=[
            pltpu.VMEM((seq_len, dim), position_embedding_table.dtype),
            pltpu.SemaphoreType.DMA((_N_CHUNKS,)),
            pltpu.SemaphoreType.DMA((_N_CHUNKS,)),
        ],
    )(position_embedding_table)


# speedup vs baseline: 1.0518x; 1.0082x over previous
import jax
import jax.numpy as jnp
from jax.experimental import pallas as pl
from jax.experimental.pallas import tpu as pltpu

_N_CHUNKS = 4


def _staged_copy(table_hbm, out_hbm, buf, sem_in, sem_out):
    rows = table_hbm.shape[0]
    chunk = rows // _N_CHUNKS

    def cin(i):
        return pltpu.make_async_copy(
            table_hbm.at[pl.ds(i * chunk, chunk)],
            buf.at[pl.ds(i * chunk, chunk)],
            sem_in.at[i],
        )

    def cout(i):
        return pltpu.make_async_copy(
            buf.at[pl.ds(i * chunk, chunk)],
            out_hbm.at[pl.ds(i * chunk, chunk)],
            sem_out.at[i],
        )

    for i in range(_N_CHUNKS):
        cin(i).start()
    for i in range(_N_CHUNKS):
        cin(i).wait()
        cout(i).start()
    for i in range(_N_CHUNKS):
        cout(i).wait()


def kernel(input_indices, position_embedding_table):
    seq_len = input_indices.shape[1]
    n_rows, dim = position_embedding_table.shape
    return pl.pallas_call(
        _staged_copy,
        in_specs=[pl.BlockSpec(memory_space=pltpu.HBM)],
        out_specs=pl.BlockSpec(memory_space=pltpu.HBM),
        out_shape=jax.ShapeDtypeStruct((seq_len, dim), position_embedding_table.dtype),
        scratch_shapes=[
            pltpu.VMEM((seq_len, dim), position_embedding_table.dtype),
            pltpu.SemaphoreType.DMA((_N_CHUNKS,)),
            pltpu.SemaphoreType.DMA((_N_CHUNKS,)),
        ],
    )(position_embedding_table)
